# baseline (speedup 1.0000x reference)
import functools

import jax
import jax.numpy as jnp
from jax.experimental import pallas as pl
from jax.experimental.pallas import tpu as pltpu

_TEMP = 20.0
_EPS = 1e-12
_BN = 32768
_HB = _BN // 2


def _tile_kernel(feat_ref, mem_a_ref, mem_b_ref, out_ref):
    feat = feat_ref[...]
    norm = jnp.sqrt(jnp.sum(feat * feat, axis=-1, keepdims=True))
    feat_n = (feat / jnp.maximum(norm, _EPS)).astype(jnp.bfloat16)
    dn = (((1,), (1,)), ((), ()))
    out_ref[:, :_HB] = jax.lax.dot_general(
        feat_n, mem_a_ref[...].astype(jnp.bfloat16), dn,
        preferred_element_type=jnp.float32) * _TEMP
    out_ref[:, _HB:] = jax.lax.dot_general(
        feat_n, mem_b_ref[...].astype(jnp.bfloat16), dn,
        preferred_element_type=jnp.float32) * _TEMP


@functools.partial(jax.jit, static_argnames=())
def kernel(feat, index, memory):
    del index
    batch, feat_dim = feat.shape
    n_data = memory.shape[0]
    grid = (pl.cdiv(n_data, _BN),)
    return pl.pallas_call(
        _tile_kernel,
        grid=grid,
        in_specs=[
            pl.BlockSpec((batch, feat_dim), lambda i: (0, 0)),
            pl.BlockSpec((_HB, feat_dim), lambda i: (2 * i, 0)),
            pl.BlockSpec((_HB, feat_dim), lambda i: (2 * i + 1, 0)),
        ],
        out_specs=pl.BlockSpec((batch, _BN), lambda i: (0, i)),
        out_shape=jax.ShapeDtypeStruct((batch, n_data), jnp.float32),
        compiler_params=pltpu.CompilerParams(
            dimension_semantics=("parallel",),
        ),
    )(feat, memory, memory)
